# baseline (speedup 1.0000x reference)
import functools

import jax
import jax.numpy as jnp
import numpy as np
from jax import lax
from jax.experimental import pallas as pl
from jax.experimental.pallas import tpu as pltpu
from jax.experimental.pallas import tpu_sc as plsc

CIRCLE = 1000
D = 64
N_HIS = 20
BATCH = 16384

NC = 2
NS = 16
NW = NC * NS
BW = BATCH // NW
NBT = BW // 128


def _tempo_enc_np(n_his, d):
    pos = np.arange(n_his, dtype=np.float64)[:, None]
    i = np.arange(d, dtype=np.float64)[None, :]
    angle = pos / np.power(10000.0, (2.0 * (i // 2)) / d)
    enc = np.zeros((n_his, d), dtype=np.float64)
    enc[:, 0::2] = np.sin(angle[:, 0::2])
    enc[:, 1::2] = np.cos(angle[:, 1::2])
    return enc.astype(np.float32)


_ENC = _tempo_enc_np(N_HIS, D)
_ENC_SPLAT = np.repeat(_ENC.reshape(N_HIS * 8, 8), 16, axis=1)


def _body(stampT_hbm, tabT_hbm, enc_hbm, out_hbm,
          tbuf, encbuf, sbufA, sbufB, stg0, stg1,
          ssemA, ssemB, osem0, osem1):
    wid = lax.axis_index("s") * NC + lax.axis_index("c")
    b0 = wid * BW
    bt0 = wid * NBT
    pltpu.sync_copy(tabT_hbm, tbuf)
    pltpu.sync_copy(enc_hbm, encbuf)
    pltpu.async_copy(stampT_hbm.at[0, pl.ds(b0, BW)], sbufA, ssemA)
    pltpu.async_copy(stampT_hbm.at[1, pl.ds(b0, BW)], sbufB, ssemB)

    def unit(h, bt, sbuf, stg, osem, first_p):
        if first_p is None:
            pltpu.make_async_copy(stg, out_hbm.at[0, :, 0, :, :], osem).wait()
        else:
            @pl.when(first_p > 0)
            def _():
                pltpu.make_async_copy(
                    stg, out_hbm.at[0, :, 0, :, :], osem).wait()
        hbase = h * 8
        idx0 = tuple(
            sbuf[pl.ds(bt * 128 + g * 16, 16)] for g in range(8))

        @plsc.parallel_loop(0, D, step=1, unroll=8, carry=idx0)
        def dbody(d, idx):
            dt = d // 8
            di = d % 8
            encv = encbuf[hbase + dt, pl.ds(di * 16, 16)]
            for g in range(8):
                row = plsc.load_gather(tbuf, [idx[g]])
                stg[dt, di, pl.ds(g * 16, 16)] = row + encv
            return tuple(v + CIRCLE for v in idx)

        pltpu.async_copy(stg, out_hbm.at[h, :, bt0 + bt, :, :], osem)

    def pair(p, carry):
        h0 = 2 * p
        pltpu.make_async_copy(
            stampT_hbm.at[0, pl.ds(b0, BW)], sbufA, ssemA).wait()
        unit(h0, 0, sbufA, stg0, osem0, p)
        unit(h0, 1, sbufA, stg1, osem1, p)
        unit(h0, 2, sbufA, stg0, osem0, None)
        unit(h0, 3, sbufA, stg1, osem1, None)

        @pl.when(p < N_HIS // 2 - 1)
        def _():
            pltpu.async_copy(
                stampT_hbm.at[h0 + 2, pl.ds(b0, BW)], sbufA, ssemA)

        pltpu.make_async_copy(
            stampT_hbm.at[1, pl.ds(b0, BW)], sbufB, ssemB).wait()
        unit(h0 + 1, 0, sbufB, stg0, osem0, None)
        unit(h0 + 1, 1, sbufB, stg1, osem1, None)
        unit(h0 + 1, 2, sbufB, stg0, osem0, None)
        unit(h0 + 1, 3, sbufB, stg1, osem1, None)

        @pl.when(p < N_HIS // 2 - 1)
        def _():
            pltpu.async_copy(
                stampT_hbm.at[h0 + 3, pl.ds(b0, BW)], sbufB, ssemB)

        return carry

    lax.fori_loop(0, N_HIS // 2, pair, 0)
    pltpu.make_async_copy(stg0, out_hbm.at[0, :, 0, :, :], osem0).wait()
    pltpu.make_async_copy(stg1, out_hbm.at[0, :, 0, :, :], osem1).wait()


_sc = functools.partial(
    pl.kernel,
    out_type=jax.ShapeDtypeStruct((N_HIS, 8, 128, 8, 128), jnp.float32),
    mesh=plsc.VectorSubcoreMesh(core_axis_name="c", subcore_axis_name="s"),
    scratch_types=[
        pltpu.VMEM((D * CIRCLE,), jnp.float32),
        pltpu.VMEM((160, 128), jnp.float32),
        pltpu.VMEM((BW,), jnp.int32),
        pltpu.VMEM((BW,), jnp.int32),
        pltpu.VMEM((8, 8, 128), jnp.float32),
        pltpu.VMEM((8, 8, 128), jnp.float32),
        pltpu.SemaphoreType.DMA,
        pltpu.SemaphoreType.DMA,
        pltpu.SemaphoreType.DMA,
        pltpu.SemaphoreType.DMA,
    ],
    compiler_params=pltpu.CompilerParams(
        use_tc_tiling_on_sc=False, needs_layout_passes=False),
)(_body)


def kernel(stamp, table):
    stampT = jnp.transpose(stamp)
    tabT = jnp.transpose(table).reshape(D * CIRCLE)
    y5 = _sc(stampT, tabT, jnp.asarray(_ENC_SPLAT))
    y = jnp.transpose(y5, (0, 1, 3, 2, 4)).reshape(N_HIS, D, BATCH)
    return jnp.transpose(y, (2, 0, 1))


# speedup vs baseline: 1.1123x; 1.1123x over previous
import functools

import jax
import jax.numpy as jnp
import numpy as np
from jax import lax
from jax.experimental import pallas as pl
from jax.experimental.pallas import tpu as pltpu
from jax.experimental.pallas import tpu_sc as plsc

CIRCLE = 1000
D = 64
N_HIS = 20
BATCH = 16384

NC = 2
NS = 16
NW = NC * NS
BW = BATCH // NW
NBT = BW // 128


def _tempo_enc_np(n_his, d):
    pos = np.arange(n_his, dtype=np.float64)[:, None]
    i = np.arange(d, dtype=np.float64)[None, :]
    angle = pos / np.power(10000.0, (2.0 * (i // 2)) / d)
    enc = np.zeros((n_his, d), dtype=np.float64)
    enc[:, 0::2] = np.sin(angle[:, 0::2])
    enc[:, 1::2] = np.cos(angle[:, 1::2])
    return enc.astype(np.float32)


_ENC = _tempo_enc_np(N_HIS, D)
_ENC_SPLAT = np.repeat(_ENC.reshape(N_HIS * 8, 8), 16, axis=1)


def _body(stampT_hbm, tabT_hbm, enc_hbm, out_hbm,
          tbuf, encbuf, sbufA, sbufB, stg0, stg1, stg2, stg3,
          ssemA, ssemB, osem0, osem1, osem2, osem3):
    wid = lax.axis_index("s") * NC + lax.axis_index("c")
    b0 = wid * BW
    bt0 = wid * NBT
    pltpu.sync_copy(tabT_hbm, tbuf)
    pltpu.sync_copy(enc_hbm, encbuf)
    pltpu.async_copy(stampT_hbm.at[0, pl.ds(b0, BW)], sbufA, ssemA)
    pltpu.async_copy(stampT_hbm.at[1, pl.ds(b0, BW)], sbufB, ssemB)

    def unit(h, bt, sbuf, stg, osem, first_p):
        if first_p is None:
            pltpu.make_async_copy(stg, out_hbm.at[0, :, 0, :, :], osem).wait()
        else:
            @pl.when(first_p > 0)
            def _():
                pltpu.make_async_copy(
                    stg, out_hbm.at[0, :, 0, :, :], osem).wait()
        hbase = h * 8
        idx0 = tuple(
            sbuf[pl.ds(bt * 128 + g * 16, 16)] for g in range(8))

        @plsc.parallel_loop(0, D, step=1, unroll=4, carry=idx0)
        def dbody(d, idx):
            dt = d // 8
            di = d % 8
            encv = encbuf[hbase + dt, pl.ds(di * 16, 16)]
            for g in range(8):
                row = plsc.load_gather(tbuf, [idx[g]])
                stg[dt, di, pl.ds(g * 16, 16)] = row + encv
            return tuple(v + CIRCLE for v in idx)

        pltpu.async_copy(stg, out_hbm.at[h, :, bt0 + bt, :, :], osem)

    stgs = None

    def pair(p, carry):
        h0 = 2 * p
        pltpu.make_async_copy(
            stampT_hbm.at[0, pl.ds(b0, BW)], sbufA, ssemA).wait()
        for bt, (stg, osem) in enumerate(stgs):
            unit(h0, bt, sbufA, stg, osem, p)

        @pl.when(p < N_HIS // 2 - 1)
        def _():
            pltpu.async_copy(
                stampT_hbm.at[h0 + 2, pl.ds(b0, BW)], sbufA, ssemA)

        pltpu.make_async_copy(
            stampT_hbm.at[1, pl.ds(b0, BW)], sbufB, ssemB).wait()
        for bt, (stg, osem) in enumerate(stgs):
            unit(h0 + 1, bt, sbufB, stg, osem, None)

        @pl.when(p < N_HIS // 2 - 1)
        def _():
            pltpu.async_copy(
                stampT_hbm.at[h0 + 3, pl.ds(b0, BW)], sbufB, ssemB)

        return carry

    stgs = [(stg0, osem0), (stg1, osem1), (stg2, osem2), (stg3, osem3)]
    lax.fori_loop(0, N_HIS // 2, pair, 0)
    for stg, osem in stgs:
        pltpu.make_async_copy(stg, out_hbm.at[0, :, 0, :, :], osem).wait()


_sc = functools.partial(
    pl.kernel,
    out_type=jax.ShapeDtypeStruct((N_HIS, 8, 128, 8, 128), jnp.float32),
    mesh=plsc.VectorSubcoreMesh(core_axis_name="c", subcore_axis_name="s"),
    scratch_types=[
        pltpu.VMEM((D * CIRCLE,), jnp.float32),
        pltpu.VMEM((160, 128), jnp.float32),
        pltpu.VMEM((BW,), jnp.int32),
        pltpu.VMEM((BW,), jnp.int32),
        pltpu.VMEM((8, 8, 128), jnp.float32),
        pltpu.VMEM((8, 8, 128), jnp.float32),
        pltpu.VMEM((8, 8, 128), jnp.float32),
        pltpu.VMEM((8, 8, 128), jnp.float32),
        pltpu.SemaphoreType.DMA,
        pltpu.SemaphoreType.DMA,
        pltpu.SemaphoreType.DMA,
        pltpu.SemaphoreType.DMA,
        pltpu.SemaphoreType.DMA,
        pltpu.SemaphoreType.DMA,
    ],
    compiler_params=pltpu.CompilerParams(
        use_tc_tiling_on_sc=False, needs_layout_passes=False),
)(_body)


def kernel(stamp, table):
    stampT = jnp.transpose(stamp)
    tabT = jnp.transpose(table).reshape(D * CIRCLE)
    y5 = _sc(stampT, tabT, jnp.asarray(_ENC_SPLAT))
    y = jnp.transpose(y5, (0, 1, 3, 2, 4)).reshape(N_HIS, D, BATCH)
    return jnp.transpose(y, (2, 0, 1))
